# baseline (speedup 1.0000x reference)
import jax
import jax.numpy as jnp
from jax import lax
from jax.experimental import pallas as pl
from jax.experimental.pallas import tpu as pltpu
from jax.experimental.pallas import tpu_sc as plsc
import functools

N = 10000
D = 256
NV = D // 16
S = 64
CH = 80
NCHUNK = N // CH
NC = 2
NS = 16
NW = NC * NS
CPW = 4


def _sc_pool_body(x_hbm, b2d_hbm, zeros_hbm, psum_hbm,
                  idx_v, x_v, acc_v, sems):
    core = lax.axis_index("c")
    sid = lax.axis_index("s")
    wid = sid * NC + core

    zcopy = pltpu.async_copy(zeros_hbm, acc_v, sems.at[2 * CPW])
    for j in range(CPW):
        c = wid + NW * j

        @pl.when(c < NCHUNK)
        def _(c=c, j=j):
            pltpu.async_copy(b2d_hbm.at[c], idx_v.at[j], sems.at[j])
            pltpu.async_copy(x_hbm.at[pl.ds(c * CH, CH)], x_v.at[j],
                             sems.at[CPW + j])

    zcopy.wait()

    for j in range(CPW):
        c = wid + NW * j

        @pl.when(c < NCHUNK)
        def _(c=c, j=j):
            pltpu.make_async_copy(b2d_hbm.at[c], idx_v.at[j],
                                  sems.at[j]).wait()
            pltpu.make_async_copy(x_hbm.at[pl.ds(c * CH, CH)], x_v.at[j],
                                  sems.at[CPW + j]).wait()

            def group_body(g, carry):
                idx16 = idx_v[j, pl.ds(g * 16, 16)]
                first = idx16[0]
                uniform = first == idx16[15]

                @pl.when(uniform)
                def _():
                    sums = [x_v[j, g * 16, pl.ds(k * 16, 16)]
                            for k in range(NV)]
                    for l in range(1, 16):
                        vals = [x_v[j, g * 16 + l, pl.ds(k * 16, 16)]
                                for k in range(NV)]
                        sums = [a + b for a, b in zip(sums, vals)]
                    for k in range(NV):
                        plsc.addupdate(acc_v.at[first, pl.ds(k * 16, 16)],
                                       sums[k])

                @pl.when(jnp.logical_not(uniform))
                def _():
                    for l in range(16):
                        s = idx16[l]
                        r = g * 16 + l
                        vals = [x_v[j, r, pl.ds(k * 16, 16)]
                                for k in range(NV)]
                        for k in range(NV):
                            plsc.addupdate(
                                acc_v.at[s, pl.ds(k * 16, 16)], vals[k])

                return carry

            lax.fori_loop(0, CH // 16, group_body, jnp.int32(0))

    pltpu.sync_copy(acc_v, psum_hbm.at[wid])


_sc_pool = functools.partial(
    pl.kernel,
    out_type=[
        jax.ShapeDtypeStruct((NW, S, D), jnp.float32),
    ],
    mesh=plsc.VectorSubcoreMesh(core_axis_name="c", subcore_axis_name="s"),
    scratch_types=[
        pltpu.VMEM((CPW, CH), jnp.int32),
        pltpu.VMEM((CPW, CH, D), jnp.float32),
        pltpu.VMEM((S, D), jnp.float32),
        pltpu.SemaphoreType.DMA((2 * CPW + 1,)),
    ],
)(_sc_pool_body)


def _combine_body(ps_ref, b_ref, o_ref):
    sums = jnp.sum(ps_ref[...], axis=0)
    seg = lax.broadcasted_iota(jnp.int32, (S, N), 0)
    onehot = (b_ref[...] == seg).astype(jnp.float32)
    cnt = jnp.sum(onehot, axis=1, keepdims=True)
    o_ref[...] = sums / jnp.maximum(cnt, 1.0)


_combine = pl.pallas_call(
    _combine_body,
    out_shape=jax.ShapeDtypeStruct((S, D), jnp.float32),
)


@jax.jit
def kernel(x, edge_index, batch):
    del edge_index
    b32 = batch.astype(jnp.int32)
    zeros = jnp.zeros((S, D), jnp.float32)
    (psum,) = _sc_pool(x, b32.reshape(NCHUNK, CH), zeros)
    return _combine(psum, b32.reshape(1, N))


# speedup vs baseline: 1.2137x; 1.2137x over previous
import jax
import jax.numpy as jnp
from jax import lax
from jax.experimental import pallas as pl
from jax.experimental.pallas import tpu as pltpu
from jax.experimental.pallas import tpu_sc as plsc
import functools

N = 10000
D = 256
NV = D // 16
S = 64
CH = 80
NCHUNK = N // CH
NC = 2
NS = 16
NW = NC * NS
CPW = 4


def _sc_pool_body(x_hbm, b2d_hbm, zeros_hbm, psum_hbm,
                  idx_v, x_v, acc_v, sems):
    core = lax.axis_index("c")
    sid = lax.axis_index("s")
    wid = sid * NC + core

    zcopy = pltpu.async_copy(zeros_hbm, acc_v, sems.at[2 * CPW])
    for j in range(CPW):
        c = wid + NW * j

        @pl.when(c < NCHUNK)
        def _(c=c, j=j):
            pltpu.async_copy(b2d_hbm.at[c], idx_v.at[j], sems.at[j])
            pltpu.async_copy(x_hbm.at[pl.ds(c * CH, CH)], x_v.at[j],
                             sems.at[CPW + j])

    zcopy.wait()

    for j in range(CPW):
        c = wid + NW * j

        @pl.when(c < NCHUNK)
        def _(c=c, j=j):
            pltpu.make_async_copy(b2d_hbm.at[c], idx_v.at[j],
                                  sems.at[j]).wait()
            pltpu.make_async_copy(x_hbm.at[pl.ds(c * CH, CH)], x_v.at[j],
                                  sems.at[CPW + j]).wait()

            def group_body(g, carry):
                idx16 = idx_v[j, pl.ds(g * 16, 16)]
                prev_s = idx16[0]
                prev = [x_v[j, g * 16, pl.ds(k * 16, 16)]
                        for k in range(NV)]
                for l in range(1, 16):
                    s = idx16[l]
                    vals = [x_v[j, g * 16 + l, pl.ds(k * 16, 16)]
                            for k in range(NV)]
                    for k in range(NV):
                        plsc.addupdate(
                            acc_v.at[prev_s, pl.ds(k * 16, 16)], prev[k])
                    prev_s, prev = s, vals
                for k in range(NV):
                    plsc.addupdate(acc_v.at[prev_s, pl.ds(k * 16, 16)],
                                   prev[k])
                return carry

            lax.fori_loop(0, CH // 16, group_body, jnp.int32(0))

    pltpu.sync_copy(acc_v, psum_hbm.at[wid])


_sc_pool = functools.partial(
    pl.kernel,
    out_type=[
        jax.ShapeDtypeStruct((NW, S, D), jnp.float32),
    ],
    mesh=plsc.VectorSubcoreMesh(core_axis_name="c", subcore_axis_name="s"),
    scratch_types=[
        pltpu.VMEM((CPW, CH), jnp.int32),
        pltpu.VMEM((CPW, CH, D), jnp.float32),
        pltpu.VMEM((S, D), jnp.float32),
        pltpu.SemaphoreType.DMA((2 * CPW + 1,)),
    ],
)(_sc_pool_body)


def _combine_body(ps_ref, b_ref, o_ref):
    sums = jnp.sum(ps_ref[...], axis=0)
    seg = lax.broadcasted_iota(jnp.int32, (S, N), 0)
    onehot = (b_ref[...] == seg).astype(jnp.float32)
    cnt = jnp.sum(onehot, axis=1, keepdims=True)
    o_ref[...] = sums / jnp.maximum(cnt, 1.0)


_combine = pl.pallas_call(
    _combine_body,
    out_shape=jax.ShapeDtypeStruct((S, D), jnp.float32),
)


@jax.jit
def kernel(x, edge_index, batch):
    del edge_index
    b32 = batch.astype(jnp.int32)
    zeros = jnp.zeros((S, D), jnp.float32)
    (psum,) = _sc_pool(x, b32.reshape(NCHUNK, CH), zeros)
    return _combine(psum, b32.reshape(1, N))


# speedup vs baseline: 1.3716x; 1.1301x over previous
import jax
import jax.numpy as jnp
from jax import lax
from jax.experimental import pallas as pl
from jax.experimental.pallas import tpu as pltpu
from jax.experimental.pallas import tpu_sc as plsc
import functools

N = 10000
D = 256
NV = D // 16
S = 64
W = 80
NG = N // 16
NC = 2
NS = 16
NW = NC * NS
SPT = S // NW


def _sc_pool_body(x_hbm, b_hbm, out_hbm, bat_v, xw_v, out_v, sems):
    core = lax.axis_index("c")
    sid = lax.axis_index("s")
    wid = sid * NC + core
    s0 = wid * SPT

    pltpu.sync_copy(b_hbm, bat_v)

    thr = [jnp.full((16,), s0 + t, jnp.int32) for t in range(SPT + 1)]
    zi = jnp.zeros((16,), jnp.int32)

    def count_body(g, accs):
        v = bat_v[pl.ds(g * 16, 16)]
        return tuple(a + jnp.minimum(jnp.maximum(t - v, 0), 1)
                     for a, t in zip(accs, thr))

    accs = lax.fori_loop(0, NG, count_body, (zi,) * (SPT + 1))

    def _hsum(a):
        t = a[0]
        for l in range(1, 16):
            t = t + a[l]
        return t

    cuts = [_hsum(a) for a in accs]

    zeros16 = jnp.zeros((16,), jnp.float32)
    for k in range(SPT):
        lo_row, hi_row = cuts[k], cuts[k + 1]
        num = hi_row - lo_row
        a0 = lo_row & ~7
        nwin = ((hi_row - a0 + (W - 1)) // W) * jnp.minimum(num, 1)

        def wbase(i):
            return pl.multiple_of(jnp.minimum(a0 + i * W, N - W), 8)

        @pl.when(num > 0)
        def _():
            pltpu.async_copy(x_hbm.at[pl.ds(wbase(0), W)], xw_v.at[0],
                             sems.at[0])

        def win_body(i, sums):
            buf = i & 1
            start = a0 + i * W
            base = wbase(i)
            pltpu.make_async_copy(x_hbm.at[pl.ds(base, W)],
                                  xw_v.at[buf], sems.at[buf]).wait()

            @pl.when(i + 1 < nwin)
            def _():
                pltpu.async_copy(x_hbm.at[pl.ds(wbase(i + 1), W)],
                                 xw_v.at[(i + 1) & 1], sems.at[(i + 1) & 1])

            lo_l = jnp.maximum(lo_row, start) - base
            hi_l = jnp.minimum(hi_row, start + W) - base

            def row_body(rl, sums2):
                vals = [xw_v[buf, rl, pl.ds(kk * 16, 16)]
                        for kk in range(NV)]
                return tuple(a + b for a, b in zip(sums2, vals))

            return lax.fori_loop(lo_l, hi_l, row_body, sums)

        sums = lax.fori_loop(0, nwin, win_body, (zeros16,) * NV)
        cnt16 = jnp.full((16,), jnp.maximum(num, 1),
                         jnp.int32).astype(jnp.float32)
        scale16 = jnp.full((16,), 1.0, jnp.float32) / cnt16
        for kk in range(NV):
            out_v[k, pl.ds(kk * 16, 16)] = sums[kk] * scale16

    pltpu.sync_copy(out_v, out_hbm.at[pl.ds(s0, SPT)])


_sc_pool = functools.partial(
    pl.kernel,
    out_type=[
        jax.ShapeDtypeStruct((S, D), jnp.float32),
    ],
    mesh=plsc.VectorSubcoreMesh(core_axis_name="c", subcore_axis_name="s"),
    scratch_types=[
        pltpu.VMEM((N,), jnp.int32),
        pltpu.VMEM((2, W, D), jnp.float32),
        pltpu.VMEM((SPT, D), jnp.float32),
        pltpu.SemaphoreType.DMA((2,)),
    ],
)(_sc_pool_body)


@jax.jit
def kernel(x, edge_index, batch):
    del edge_index
    (out,) = _sc_pool(x, batch.astype(jnp.int32))
    return out


# speedup vs baseline: 1.4235x; 1.0378x over previous
import jax
import jax.numpy as jnp
from jax import lax
from jax.experimental import pallas as pl
from jax.experimental.pallas import tpu as pltpu
from jax.experimental.pallas import tpu_sc as plsc
import functools

N = 10000
D = 256
NV = D // 16
S = 64
W = 80
NG = N // 16
NC = 2
NS = 16
NW = NC * NS
SPT = S // NW


def _sc_pool_body(x_hbm, b_hbm, out_hbm, bat_v, xw_v, out_v, sems):
    core = lax.axis_index("c")
    sid = lax.axis_index("s")
    wid = sid * NC + core
    s0 = wid * SPT

    pltpu.sync_copy(b_hbm, bat_v)

    thr = [jnp.full((16,), s0 + t, jnp.int32) for t in range(SPT + 1)]
    zi = jnp.zeros((16,), jnp.int32)

    def count_body(g5, accs):
        for u in range(5):
            v = bat_v[pl.ds((g5 * 5 + u) * 16, 16)]
            accs = tuple(a + jnp.minimum(jnp.maximum(t - v, 0), 1)
                         for a, t in zip(accs, thr))
        return accs

    accs = lax.fori_loop(0, NG // 5, count_body, (zi,) * (SPT + 1))

    def _hsum(a):
        t = a[0]
        for l in range(1, 16):
            t = t + a[l]
        return t

    cuts = [_hsum(a) for a in accs]

    zeros16 = jnp.zeros((16,), jnp.float32)
    for k in range(SPT):
        lo_row, hi_row = cuts[k], cuts[k + 1]
        num = hi_row - lo_row
        a0 = lo_row & ~7
        nwin = ((hi_row - a0 + (W - 1)) // W) * jnp.minimum(num, 1)

        def wbase(i):
            return pl.multiple_of(jnp.minimum(a0 + i * W, N - W), 8)

        @pl.when(num > 0)
        def _():
            pltpu.async_copy(x_hbm.at[pl.ds(wbase(0), W)], xw_v.at[0],
                             sems.at[0])

        def win_body(i, sums):
            buf = i & 1
            start = a0 + i * W
            base = wbase(i)
            pltpu.make_async_copy(x_hbm.at[pl.ds(base, W)],
                                  xw_v.at[buf], sems.at[buf]).wait()

            @pl.when(i + 1 < nwin)
            def _():
                pltpu.async_copy(x_hbm.at[pl.ds(wbase(i + 1), W)],
                                 xw_v.at[(i + 1) & 1], sems.at[(i + 1) & 1])

            lo_l = jnp.maximum(lo_row, start) - base
            hi_l = jnp.minimum(hi_row, start + W) - base

            def row_body(rl, sums2):
                vals = [xw_v[buf, rl, pl.ds(kk * 16, 16)]
                        for kk in range(NV)]
                return tuple(a + b for a, b in zip(sums2, vals))

            return lax.fori_loop(lo_l, hi_l, row_body, sums)

        sums = lax.fori_loop(0, nwin, win_body, (zeros16,) * NV)
        cnt16 = jnp.full((16,), jnp.maximum(num, 1),
                         jnp.int32).astype(jnp.float32)
        scale16 = jnp.full((16,), 1.0, jnp.float32) / cnt16
        for kk in range(NV):
            out_v[k, pl.ds(kk * 16, 16)] = sums[kk] * scale16

    pltpu.sync_copy(out_v, out_hbm.at[pl.ds(s0, SPT)])


_sc_pool = functools.partial(
    pl.kernel,
    out_type=[
        jax.ShapeDtypeStruct((S, D), jnp.float32),
    ],
    mesh=plsc.VectorSubcoreMesh(core_axis_name="c", subcore_axis_name="s"),
    scratch_types=[
        pltpu.VMEM((N,), jnp.int32),
        pltpu.VMEM((2, W, D), jnp.float32),
        pltpu.VMEM((SPT, D), jnp.float32),
        pltpu.SemaphoreType.DMA((2,)),
    ],
)(_sc_pool_body)


@jax.jit
def kernel(x, edge_index, batch):
    del edge_index
    (out,) = _sc_pool(x, batch.astype(jnp.int32))
    return out
